# baseline (speedup 1.0000x reference)
import functools
import jax
import jax.numpy as jnp
from jax import lax
from jax.experimental import pallas as pl
from jax.experimental.pallas import tpu as pltpu
from jax.experimental.pallas import tpu_sc as plsc

BATCH = 16384
DIM = 64

_info = plsc.get_sparse_core_info()
_NC, _NS = _info.num_cores, _info.num_subcores
_NW = _NC * _NS
_BPW = BATCH // _NW
_NCHUNK = 4
_C = _BPW // _NCHUNK


def _sc_gather(h2, r2, t2, ent2, rele2, reln2):
    mesh = plsc.VectorSubcoreMesh(core_axis_name="c", subcore_axis_name="s")
    out = jax.ShapeDtypeStruct((BATCH, 2 * DIM), jnp.float32)

    @functools.partial(
        pl.kernel,
        mesh=mesh,
        out_type=[out, out, out, out],
        scratch_types=(
            [pltpu.VMEM((_C,), jnp.int32) for _ in range(3 * _NCHUNK)]
            + [
                pltpu.VMEM((_C, 2 * DIM), jnp.float32),
                pltpu.VMEM((_C, 2 * DIM), jnp.float32),
                pltpu.VMEM((_C, 2 * DIM), jnp.float32),
                pltpu.VMEM((_C, 2 * DIM), jnp.float32),
                pltpu.SemaphoreType.DMA,
            ]
        ),
    )
    def k(h_hbm, r_hbm, t_hbm, ent_hbm, rele_hbm, reln_hbm,
          ho_hbm, ro_hbm, to_hbm, wo_hbm,
          *scratch):
        idx = scratch[: 3 * _NCHUNK]
        hb, rb, tb, wb, sem = scratch[3 * _NCHUNK:]
        wid = lax.axis_index("s") * _NC + lax.axis_index("c")
        base = wid * _BPW
        for c in range(_NCHUNK):
            off = base + c * _C
            ih, ir, it = idx[3 * c: 3 * c + 3]
            pltpu.sync_copy(h_hbm.at[pl.ds(off, _C)], ih)
            pltpu.sync_copy(r_hbm.at[pl.ds(off, _C)], ir)
            pltpu.sync_copy(t_hbm.at[pl.ds(off, _C)], it)
            cps = [
                pltpu.async_copy(ent_hbm.at[ih], hb, sem),
                pltpu.async_copy(rele_hbm.at[ir], rb, sem),
                pltpu.async_copy(ent_hbm.at[it], tb, sem),
                pltpu.async_copy(reln_hbm.at[ir], wb, sem),
            ]
            for cp in cps:
                cp.wait()
            pltpu.sync_copy(hb, ho_hbm.at[pl.ds(off, _C)])
            pltpu.sync_copy(rb, ro_hbm.at[pl.ds(off, _C)])
            pltpu.sync_copy(tb, to_hbm.at[pl.ds(off, _C)])
            pltpu.sync_copy(wb, wo_hbm.at[pl.ds(off, _C)])

    return k(h2, r2, t2, ent2, rele2, reln2)


def _sel(rows_ref, par):
    lo = rows_ref[:, :DIM]
    hi = rows_ref[:, DIM:]
    return jnp.where(par, hi, lo)


_TCB = 2048


def _tc_score_body(hp_ref, rp_ref, tp_ref, h2_ref, r2_ref, t2_ref,
                   w2_ref, o_ref):
    hpar = hp_ref[...] == 1
    rpar = rp_ref[...] == 1
    tpar = tp_ref[...] == 1
    he = _sel(h2_ref, hpar)
    re = _sel(r2_ref, rpar)
    te = _sel(t2_ref, tpar)
    w = _sel(w2_ref, rpar)
    d = he + re - te
    m = jnp.maximum(jnp.sqrt(jnp.sum(w * w, axis=1, keepdims=True)), 1e-12)
    wn = w / m
    s1 = jnp.sum(d * wn, axis=1, keepdims=True)
    proj = d - s1 * wn
    o_ref[...] = jnp.sqrt(jnp.sum(proj * proj, axis=1))


def _tc_score(hpar, rpar, tpar, h2_rows, r2_rows, t2_rows, w2_rows):
    npar = pl.BlockSpec((_TCB, 1), lambda i: (i, 0))
    nrow = pl.BlockSpec((_TCB, 2 * DIM), lambda i: (i, 0))
    return pl.pallas_call(
        _tc_score_body,
        grid=(BATCH // _TCB,),
        in_specs=[npar, npar, npar, nrow, nrow, nrow, nrow],
        out_specs=pl.BlockSpec((_TCB,), lambda i: (i,)),
        out_shape=jax.ShapeDtypeStruct((BATCH,), jnp.float32),
    )(hpar, rpar, tpar, h2_rows, r2_rows, t2_rows, w2_rows)


@jax.jit
def kernel(h, r, t, ent_emb, rel_emb, rel_norm):
    ent2 = ent_emb.reshape(ent_emb.shape[0] // 2, 2 * DIM)
    rele2 = rel_emb.reshape(rel_emb.shape[0] // 2, 2 * DIM)
    reln2 = rel_norm.reshape(rel_norm.shape[0] // 2, 2 * DIM)
    h2, hpar = h // 2, (h % 2).reshape(BATCH, 1)
    r2, rpar = r // 2, (r % 2).reshape(BATCH, 1)
    t2, tpar = t // 2, (t % 2).reshape(BATCH, 1)
    hr, rr, tr, wr = _sc_gather(h2, r2, t2, ent2, rele2, reln2)
    return _tc_score(hpar, rpar, tpar, hr, rr, tr, wr)


# speedup vs baseline: 1.8869x; 1.8869x over previous
import functools
import jax
import jax.numpy as jnp
from jax import lax
from jax.experimental import pallas as pl
from jax.experimental.pallas import tpu as pltpu
from jax.experimental.pallas import tpu_sc as plsc

BATCH = 16384
DIM = 64

_info = plsc.get_sparse_core_info()
_NC, _NS = _info.num_cores, _info.num_subcores
_NW = _NC * _NS
_BPW = BATCH // _NW
_C = 32
_NCH = _BPW // _C


def _score_kernel(h, r, t, ent3, rel2, reln2):
    mesh = plsc.VectorSubcoreMesh(core_axis_name="c", subcore_axis_name="s")

    @functools.partial(
        pl.kernel,
        mesh=mesh,
        out_type=jax.ShapeDtypeStruct((BATCH,), jnp.float32),
        compiler_params=pltpu.CompilerParams(needs_layout_passes=False),
        scratch_types=[
            pltpu.VMEM((_BPW + 16,), jnp.int32),
            pltpu.VMEM((_BPW + 16,), jnp.int32),
            pltpu.VMEM((_BPW,), jnp.int32),
            pltpu.VMEM((_BPW,), jnp.int32),
            pltpu.VMEM((_C, 8, DIM), jnp.float32),
            pltpu.VMEM((_C, 8, DIM), jnp.float32),
            pltpu.VMEM((_C, 2 * DIM), jnp.float32),
            pltpu.VMEM((_C, 2 * DIM), jnp.float32),
            pltpu.VMEM((_BPW,), jnp.float32),
            pltpu.SemaphoreType.DMA,
            pltpu.SemaphoreType.DMA,
        ],
    )
    def k(h_hbm, r_hbm, t_hbm, ent_hbm, rel_hbm, reln_hbm, o_hbm,
          ihv, itv, irv, ir2, hb, tb, rb, wb, acc, sem, rsem):
        wid = lax.axis_index("s") * _NC + lax.axis_index("c")
        base = wid * _BPW
        pltpu.sync_copy(h_hbm.at[pl.ds(base, _BPW)], ihv.at[pl.ds(0, _BPW)])
        pltpu.sync_copy(t_hbm.at[pl.ds(base, _BPW)], itv.at[pl.ds(0, _BPW)])
        pltpu.sync_copy(r_hbm.at[pl.ds(base, _BPW)], irv)
        for j in range(_BPW // 16):
            sl = pl.ds(j * 16, 16)
            ir2[sl] = lax.shift_right_logical(irv[sl], 1)

        for c in range(_NCH):
            coff = c * _C

            def fire(i, _):
                eh = ihv[pl.ds(coff + i, 16)][0]
                et = itv[pl.ds(coff + i, 16)][0]
                pltpu.async_copy(
                    ent_hbm.at[pl.ds(lax.shift_right_logical(eh, 3), 1)],
                    hb.at[pl.ds(i, 1)], sem)
                pltpu.async_copy(
                    ent_hbm.at[pl.ds(lax.shift_right_logical(et, 3), 1)],
                    tb.at[pl.ds(i, 1)], sem)
                return ()

            lax.fori_loop(0, _C, fire, ())
            rcp = pltpu.async_copy(rel_hbm.at[ir2.at[pl.ds(coff, _C)]],
                                   rb, rsem)
            wcp = pltpu.async_copy(reln_hbm.at[ir2.at[pl.ds(coff, _C)]],
                                   wb, rsem)

            def drain(i, _):
                pltpu.make_async_copy(ent_hbm.at[pl.ds(0, 1)],
                                      hb.at[pl.ds(i, 1)], sem).wait()
                pltpu.make_async_copy(ent_hbm.at[pl.ds(0, 1)],
                                      tb.at[pl.ds(i, 1)], sem).wait()
                return ()

            lax.fori_loop(0, _C, drain, ())
            rcp.wait()
            wcp.wait()

            for g in range(_C // 16):
                sl = pl.ds(coff + g * 16, 16)
                hsub = lax.rem(ihv[sl], 8)
                tsub = lax.rem(itv[sl], 8)
                rpar = lax.mul(lax.rem(irv[sl], 2), DIM)
                elem = lax.iota(jnp.int32, 16) + g * 16
                zero = jnp.zeros((16,), jnp.float32)

                def dims(d, carry):
                    sxx, sxw, sww = carry
                    d16 = jnp.full((16,), d, jnp.int32)
                    hd = plsc.load_gather(hb, [elem, hsub, d16])
                    td = plsc.load_gather(tb, [elem, tsub, d16])
                    rd = plsc.load_gather(rb, [elem, d16 + rpar])
                    wd = plsc.load_gather(wb, [elem, d16 + rpar])
                    x = hd + rd - td
                    return (sxx + x * x, sxw + x * wd, sww + wd * wd)

                sxx, sxw, sww = lax.fori_loop(
                    0, DIM, dims, (zero, zero, zero))
                m2 = jnp.maximum(sww, 1e-24)
                val = jnp.maximum(sxx - (sxw * sxw) / m2, 0.0)
                bits = plsc.bitcast(val, jnp.int32)
                y = plsc.bitcast(
                    jnp.int32(0x5F3759DF) - lax.shift_right_logical(bits, 1),
                    jnp.float32)
                half = val * 0.5
                for _ in range(3):
                    y = y * (1.5 - half * y * y)
                acc[sl] = val * y

        pltpu.sync_copy(acc, o_hbm.at[pl.ds(base, _BPW)])

    return k(h, r, t, ent3, rel2, reln2)


@jax.jit
def kernel(h, r, t, ent_emb, rel_emb, rel_norm):
    ent3 = ent_emb.reshape(ent_emb.shape[0] // 8, 8, DIM)
    rel2 = rel_emb.reshape(rel_emb.shape[0] // 2, 2 * DIM)
    reln2 = rel_norm.reshape(rel_norm.shape[0] // 2, 2 * DIM)
    return _score_kernel(h, r, t, ent3, rel2, reln2)


# speedup vs baseline: 2.2281x; 1.1809x over previous
import functools
import jax
import jax.numpy as jnp
from jax import lax
from jax.experimental import pallas as pl
from jax.experimental.pallas import tpu as pltpu
from jax.experimental.pallas import tpu_sc as plsc

BATCH = 16384
DIM = 64

_info = plsc.get_sparse_core_info()
_NC, _NS = _info.num_cores, _info.num_subcores
_NW = _NC * _NS
_BPW = BATCH // _NW
_C = 16
_NCH = _BPW // _C


def _score_kernel(h, r, t, ent3, rel2, reln2):
    mesh = plsc.VectorSubcoreMesh(core_axis_name="c", subcore_axis_name="s")

    @functools.partial(
        pl.kernel,
        mesh=mesh,
        out_type=jax.ShapeDtypeStruct((BATCH,), jnp.float32),
        compiler_params=pltpu.CompilerParams(needs_layout_passes=False),
        scratch_types=[
            pltpu.VMEM((_BPW + 16,), jnp.int32),
            pltpu.VMEM((_BPW + 16,), jnp.int32),
            pltpu.VMEM((_BPW,), jnp.int32),
            pltpu.VMEM((_BPW,), jnp.int32),
            pltpu.VMEM((2, _C, 8, DIM), jnp.float32),
            pltpu.VMEM((2, _C, 8, DIM), jnp.float32),
            pltpu.VMEM((2, _C, 2 * DIM), jnp.float32),
            pltpu.VMEM((2, _C, 2 * DIM), jnp.float32),
            pltpu.VMEM((_BPW,), jnp.float32),
            pltpu.SemaphoreType.DMA,
            pltpu.SemaphoreType.DMA,
            pltpu.SemaphoreType.DMA,
            pltpu.SemaphoreType.DMA,
        ],
    )
    def k(h_hbm, r_hbm, t_hbm, ent_hbm, rel_hbm, reln_hbm, o_hbm,
          ihv, itv, irv, ir2, hbb, tbb, rbb, wbb, acc, s0, s1, q0, q1):
        wid = lax.axis_index("s") * _NC + lax.axis_index("c")
        base = wid * _BPW
        pltpu.sync_copy(h_hbm.at[pl.ds(base, _BPW)], ihv.at[pl.ds(0, _BPW)])
        pltpu.sync_copy(t_hbm.at[pl.ds(base, _BPW)], itv.at[pl.ds(0, _BPW)])
        pltpu.sync_copy(r_hbm.at[pl.ds(base, _BPW)], irv)
        for j in range(_BPW // 16):
            sl = pl.ds(j * 16, 16)
            ir2[sl] = lax.shift_right_logical(irv[sl], 1)

        sems = (s0, s1)
        rsems = (q0, q1)

        def fire(c):
            p = c % 2
            hb, tb = hbb.at[p], tbb.at[p]
            sem = sems[p]
            coff = c * _C

            def body(i, _):
                i2 = i * 2
                eh0 = ihv[pl.ds(coff + i2, 16)][0]
                et0 = itv[pl.ds(coff + i2, 16)][0]
                eh1 = ihv[pl.ds(coff + i2 + 1, 16)][0]
                et1 = itv[pl.ds(coff + i2 + 1, 16)][0]
                pltpu.async_copy(
                    ent_hbm.at[pl.ds(lax.shift_right_logical(eh0, 3), 1)],
                    hb.at[pl.ds(i2, 1)], sem)
                pltpu.async_copy(
                    ent_hbm.at[pl.ds(lax.shift_right_logical(et0, 3), 1)],
                    tb.at[pl.ds(i2, 1)], sem)
                pltpu.async_copy(
                    ent_hbm.at[pl.ds(lax.shift_right_logical(eh1, 3), 1)],
                    hb.at[pl.ds(i2 + 1, 1)], sem)
                pltpu.async_copy(
                    ent_hbm.at[pl.ds(lax.shift_right_logical(et1, 3), 1)],
                    tb.at[pl.ds(i2 + 1, 1)], sem)
                return ()

            lax.fori_loop(0, _C // 2, body, ())
            pltpu.async_copy(rel_hbm.at[ir2.at[pl.ds(coff, _C)]],
                             rbb.at[p], rsems[p])
            pltpu.async_copy(reln_hbm.at[ir2.at[pl.ds(coff, _C)]],
                             wbb.at[p], rsems[p])

        def drain(c):
            p = c % 2
            pltpu.make_async_copy(ent_hbm.at[pl.ds(0, _C)],
                                  hbb.at[p], sems[p]).wait()
            pltpu.make_async_copy(ent_hbm.at[pl.ds(0, _C)],
                                  tbb.at[p], sems[p]).wait()
            pltpu.make_async_copy(rel_hbm.at[ir2.at[pl.ds(0, _C)]],
                                  rbb.at[p], rsems[p]).wait()
            pltpu.make_async_copy(reln_hbm.at[ir2.at[pl.ds(0, _C)]],
                                  wbb.at[p], rsems[p]).wait()

        def compute(c):
            p = c % 2
            hb, tb, rb, wb = hbb.at[p], tbb.at[p], rbb.at[p], wbb.at[p]
            coff = c * _C
            for g in range(_C // 16):
                sl = pl.ds(coff + g * 16, 16)
                hsub = lax.rem(ihv[sl], 8)
                tsub = lax.rem(itv[sl], 8)
                rpar = lax.mul(lax.rem(irv[sl], 2), DIM)
                elem = lax.iota(jnp.int32, 16) + g * 16
                zero = jnp.zeros((16,), jnp.float32)

                def dims(dd, carry):
                    sxx, sxw, sww = carry
                    d0 = dd * 8
                    for u in range(8):
                        d16 = jnp.full((16,), d0 + u, jnp.int32)
                        rw = d16 + rpar
                        hd = plsc.load_gather(hb, [elem, hsub, d16])
                        td = plsc.load_gather(tb, [elem, tsub, d16])
                        rd = plsc.load_gather(rb, [elem, rw])
                        wd = plsc.load_gather(wb, [elem, rw])
                        x = hd + rd - td
                        sxx = sxx + x * x
                        sxw = sxw + x * wd
                        sww = sww + wd * wd
                    return (sxx, sxw, sww)

                sxx, sxw, sww = lax.fori_loop(
                    0, DIM // 8, dims, (zero, zero, zero))
                m2 = jnp.maximum(sww, 1e-24)
                val = jnp.maximum(sxx - (sxw * sxw) / m2, 0.0)
                bits = plsc.bitcast(val, jnp.int32)
                y = plsc.bitcast(
                    jnp.int32(0x5F3759DF) - lax.shift_right_logical(bits, 1),
                    jnp.float32)
                half = val * 0.5
                for _ in range(3):
                    y = y * (1.5 - half * y * y)
                acc[sl] = val * y

        fire(0)
        for c in range(_NCH):
            if c + 1 < _NCH:
                fire(c + 1)
            drain(c)
            compute(c)

        pltpu.sync_copy(acc, o_hbm.at[pl.ds(base, _BPW)])

    return k(h, r, t, ent3, rel2, reln2)


@jax.jit
def kernel(h, r, t, ent_emb, rel_emb, rel_norm):
    ent3 = ent_emb.reshape(ent_emb.shape[0] // 8, 8, DIM)
    rel2 = rel_emb.reshape(rel_emb.shape[0] // 2, 2 * DIM)
    reln2 = rel_norm.reshape(rel_norm.shape[0] // 2, 2 * DIM)
    return _score_kernel(h, r, t, ent3, rel2, reln2)
